# baseline (speedup 1.0000x reference)
import functools

import jax
import jax.numpy as jnp
from jax import lax
from jax.experimental import pallas as pl
from jax.experimental.pallas import tpu as pltpu
from jax.experimental.pallas import tpu_sc as plsc

N = 10000
E = 320000
NG = 64
D = 128
NCLS = 10

NC = 2
NS = 16
NW = NC * NS
EPW = E // NW
CHUNK = 125
NCHUNK = EPW // CHUNK
GRP = 40
NPAD = 10240
RPT = NPAD // NS

_sc_mesh = functools.partial(
    plsc.VectorSubcoreMesh, core_axis_name="c", subcore_axis_name="s")


def _deg_body(ones_s_hbm, ones_c_hbm, dst_hbm, out_hbm, dst_v, rows_v, acc,
              sem):
    cid = lax.axis_index("c")
    sid = lax.axis_index("s")
    wid = sid * NC + cid

    pltpu.sync_copy(ones_s_hbm, acc.at[pl.ds(sid * RPT, RPT)])
    pltpu.sync_copy(ones_c_hbm, rows_v)
    plsc.subcore_barrier()

    def group(g, _):
        g0 = pl.multiple_of(g * GRP, GRP)
        pltpu.sync_copy(dst_hbm.at[wid, pl.ds(g0, GRP)], dst_v)

        def fire(j, _):
            pltpu.async_copy(rows_v, acc.at[dst_v.at[j]], sem, add=True)
            return _
        lax.fori_loop(0, GRP, fire, None, unroll=False)

        def drain(j, _):
            pltpu.make_async_copy(rows_v, acc.at[dst_v.at[j]], sem).wait()
            return _
        lax.fori_loop(0, GRP, drain, None, unroll=False)
        return _
    lax.fori_loop(0, NCHUNK // GRP, group, None, unroll=False)

    plsc.subcore_barrier()
    pltpu.sync_copy(acc.at[pl.ds(sid * RPT, RPT)],
                    out_hbm.at[cid, pl.ds(sid * RPT, RPT)])


DW = 32

_deg_kernel = pl.kernel(
    _deg_body,
    out_type=jax.ShapeDtypeStruct((NC, NPAD, DW), jnp.float32),
    mesh=_sc_mesh(),
    scratch_types=[
        pltpu.VMEM((GRP, CHUNK), jnp.int32),
        pltpu.VMEM((CHUNK, DW), jnp.float32),
        pltpu.VMEM_SHARED((NPAD, DW), jnp.float32),
        pltpu.SemaphoreType.DMA,
    ],
)


def _agg_body(table_hbm, src_hbm, dst_hbm, out_hbm, src_v, dst_v,
              rows0, rows1, acc, sem0, sem1, sems0, sems1):
    cid = lax.axis_index("c")
    sid = lax.axis_index("s")
    wid = sid * NC + cid

    pltpu.sync_copy(src_hbm.at[wid, pl.ds(0, GRP)], src_v)
    pltpu.sync_copy(dst_hbm.at[wid, pl.ds(0, GRP)], dst_v)
    pltpu.async_copy(table_hbm.at[src_v.at[0]], rows0, sem0)
    pltpu.sync_copy(table_hbm.at[pl.ds(sid * RPT, RPT)],
                    acc.at[pl.ds(sid * RPT, RPT)])
    plsc.subcore_barrier()

    def group(g, _):
        @pl.when(g > 0)
        def _():
            g0 = pl.multiple_of(g * GRP, GRP)
            pltpu.sync_copy(src_hbm.at[wid, pl.ds(g0, GRP)], src_v)
            pltpu.sync_copy(dst_hbm.at[wid, pl.ds(g0, GRP)], dst_v)
            pltpu.async_copy(table_hbm.at[src_v.at[0]], rows0, sem0)

        def pair(k, _):
            j0 = 2 * k
            pltpu.async_copy(table_hbm.at[src_v.at[j0 + 1]], rows1, sem1)
            pltpu.make_async_copy(table_hbm.at[src_v.at[j0]], rows0,
                                  sem0).wait()
            pltpu.async_copy(rows0, acc.at[dst_v.at[j0]], sems0, add=True)

            pltpu.make_async_copy(table_hbm.at[src_v.at[j0 + 1]], rows1,
                                  sem1).wait()
            pltpu.async_copy(rows1, acc.at[dst_v.at[j0 + 1]], sems1, add=True)

            pltpu.make_async_copy(rows0, acc.at[dst_v.at[j0]], sems0).wait()

            @pl.when(k < GRP // 2 - 1)
            def _():
                pltpu.async_copy(table_hbm.at[src_v.at[j0 + 2]], rows0, sem0)

            pltpu.make_async_copy(rows1, acc.at[dst_v.at[j0 + 1]],
                                  sems1).wait()
            return _
        lax.fori_loop(0, GRP // 2, pair, None, unroll=False)
        return _
    lax.fori_loop(0, NCHUNK // GRP, group, None, unroll=False)

    plsc.subcore_barrier()
    pltpu.sync_copy(acc.at[pl.ds(sid * RPT, RPT)],
                    out_hbm.at[cid, pl.ds(sid * RPT, RPT)])


_agg_kernel = pl.kernel(
    _agg_body,
    out_type=jax.ShapeDtypeStruct((NC, NPAD, D), jnp.float32),
    mesh=_sc_mesh(),
    scratch_types=[
        pltpu.VMEM((GRP, CHUNK), jnp.int32),
        pltpu.VMEM((GRP, CHUNK), jnp.int32),
        pltpu.VMEM((CHUNK, D), jnp.float32),
        pltpu.VMEM((CHUNK, D), jnp.float32),
        pltpu.VMEM_SHARED((NPAD, D), jnp.float32),
        pltpu.SemaphoreType.DMA,
        pltpu.SemaphoreType.DMA,
        pltpu.SemaphoreType.DMA,
        pltpu.SemaphoreType.DMA,
    ],
)


BN = 640


def _mm1_body(x_ref, w_ref, b_ref, d0_ref, d1_ref, s_ref, dinv_ref):
    i = pl.program_id(0)
    col = lax.broadcasted_iota(jnp.int32, (BN, D), 1)
    t = jnp.where(col == 0, 0.0, x_ref[...])
    h = jnp.dot(t, w_ref[...], preferred_element_type=jnp.float32) + b_ref[...]
    row = i * BN + lax.broadcasted_iota(jnp.int32, (BN, D), 0)
    h = jnp.where((col == 0) | (row >= N), 0.0, h)
    deg = d0_ref[:, 0:1] + d1_ref[:, 0:1] - 1.0
    dinv = lax.rsqrt(deg)
    s_ref[...] = h * dinv
    dinv_ref[...] = dinv


def _tc_mm1(xp, W1, b1, deg0, deg1):
    return pl.pallas_call(
        _mm1_body,
        grid=(NPAD // BN,),
        in_specs=[
            pl.BlockSpec((BN, D), lambda i: (i, 0)),
            pl.BlockSpec((D, D), lambda i: (0, 0)),
            pl.BlockSpec((1, D), lambda i: (0, 0)),
            pl.BlockSpec((BN, DW), lambda i: (i, 0)),
            pl.BlockSpec((BN, DW), lambda i: (i, 0)),
        ],
        out_specs=[
            pl.BlockSpec((BN, D), lambda i: (i, 0)),
            pl.BlockSpec((BN, 1), lambda i: (i, 0)),
        ],
        out_shape=[
            jax.ShapeDtypeStruct((NPAD, D), jnp.float32),
            jax.ShapeDtypeStruct((NPAD, 1), jnp.float32),
        ],
    )(xp, W1, b1, deg0, deg1)


def _mm2_body(p0_ref, p1_ref, s_ref, dinv_ref, w_ref, b_ref, out_ref):
    dinv = dinv_ref[...]
    f32 = lambda r: r[...].astype(jnp.float32)
    a = (f32(p0_ref) + f32(p1_ref) - f32(s_ref)) * dinv
    r = jnp.maximum(a, 0.0)
    h = jnp.dot(r, w_ref[...], preferred_element_type=jnp.float32) + b_ref[...]
    col = lax.broadcasted_iota(jnp.int32, (BN, D), 1)
    h = jnp.where(col == 0, 0.0, h)
    out_ref[...] = h * dinv


def _tc_mm2(p0, p1, s1, dinv, W2, b2):
    return pl.pallas_call(
        _mm2_body,
        grid=(NPAD // BN,),
        in_specs=[
            pl.BlockSpec((BN, D), lambda i: (i, 0)),
            pl.BlockSpec((BN, D), lambda i: (i, 0)),
            pl.BlockSpec((BN, D), lambda i: (i, 0)),
            pl.BlockSpec((BN, 1), lambda i: (i, 0)),
            pl.BlockSpec((D, D), lambda i: (0, 0)),
            pl.BlockSpec((1, D), lambda i: (0, 0)),
        ],
        out_specs=pl.BlockSpec((BN, D), lambda i: (i, 0)),
        out_shape=jax.ShapeDtypeStruct((NPAD, D), jnp.float32),
    )(p0, p1, s1, dinv, W2, b2)


def _pool_body(p0_ref, p1_ref, s_ref, dinv_ref, batch_ref, w_ref, b_ref,
               out_ref, pooled_acc, cnt_acc):
    i = pl.program_id(0)
    f32 = lambda r: r[...].astype(jnp.float32)
    a = (f32(p0_ref) + f32(p1_ref) - f32(s_ref)) * dinv_ref[...]
    f = jnp.maximum(a, 0.0)
    onehot = (batch_ref[0] == lax.broadcasted_iota(jnp.int32, (NG, BN), 0)
              ).astype(jnp.float32)
    part = jnp.dot(onehot, f, preferred_element_type=jnp.float32)
    cnt = jnp.sum(onehot, axis=1, keepdims=True)

    @pl.when(i == 0)
    def _():
        pooled_acc[...] = jnp.zeros_like(pooled_acc)
        cnt_acc[...] = jnp.zeros_like(cnt_acc)

    pooled_acc[...] += part
    cnt_acc[...] += cnt

    @pl.when(i == (NPAD // BN) - 1)
    def _():
        pooled = pooled_acc[...] / jnp.maximum(cnt_acc[...], 1.0)
        logits = jnp.dot(pooled, w_ref[...],
                         preferred_element_type=jnp.float32) + b_ref[...]
        m = jnp.max(logits, axis=-1, keepdims=True)
        lse = jnp.log(jnp.sum(jnp.exp(logits - m), axis=-1, keepdims=True)) + m
        out_ref[...] = logits - lse


def _tc_pool(p0, p1, s2, dinv, batch3, W4, b4):
    return pl.pallas_call(
        _pool_body,
        grid=(NPAD // BN,),
        in_specs=[
            pl.BlockSpec((BN, D), lambda i: (i, 0)),
            pl.BlockSpec((BN, D), lambda i: (i, 0)),
            pl.BlockSpec((BN, D), lambda i: (i, 0)),
            pl.BlockSpec((BN, 1), lambda i: (i, 0)),
            pl.BlockSpec((1, 1, BN), lambda i: (i, 0, 0)),
            pl.BlockSpec((D, NCLS), lambda i: (0, 0)),
            pl.BlockSpec((1, NCLS), lambda i: (0, 0)),
        ],
        out_specs=pl.BlockSpec((NG, NCLS), lambda i: (0, 0)),
        out_shape=jax.ShapeDtypeStruct((NG, NCLS), jnp.float32),
        scratch_shapes=[
            pltpu.VMEM((NG, D), jnp.float32),
            pltpu.VMEM((NG, 1), jnp.float32),
        ],
    )(p0, p1, s2, dinv, batch3, W4, b4)


def kernel(x, edge_index, batch, W1, b1, W2, b2, W4, b4):
    e3 = edge_index.reshape(2, NW, NCHUNK, CHUNK)
    src3, dst3 = e3[0], e3[1]
    batch3 = jnp.pad(batch, (0, NPAD - N), constant_values=NG
                     ).reshape(NPAD // BN, 1, BN)
    ones_s = jnp.ones((RPT, DW), jnp.float32)
    ones_c = jnp.ones((CHUNK, DW), jnp.float32)

    degp = _deg_kernel(ones_s, ones_c, dst3)
    s1, dinv = _tc_mm1(x, W1, b1.reshape(1, D), degp[0], degp[1])
    p = _agg_kernel(s1, src3, dst3)
    s2 = _tc_mm2(p[0], p[1], s1, dinv, W2, b2.reshape(1, D))
    p2 = _agg_kernel(s2, src3, dst3)
    return _tc_pool(p2[0], p2[1], s2, dinv, batch3, W4,
                    b4.reshape(1, NCLS))


# speedup vs baseline: 1.1944x; 1.1944x over previous
import functools

import jax
import jax.numpy as jnp
from jax import lax
from jax.experimental import pallas as pl
from jax.experimental.pallas import tpu as pltpu
from jax.experimental.pallas import tpu_sc as plsc

N = 10000
E = 320000
NG = 64
D = 128
NCLS = 10

NC = 2
NS = 16
NW = NC * NS
EPW = E // NW
CHUNK = 125
NCHUNK = EPW // CHUNK
GRP = 40
NPAD = 10240
RPT = NPAD // NS

_sc_mesh = functools.partial(
    plsc.VectorSubcoreMesh, core_axis_name="c", subcore_axis_name="s")


def _deg_body(ones_s_hbm, ones_c_hbm, dst_hbm, out_hbm, dst_v, rows_v, acc,
              sem):
    cid = lax.axis_index("c")
    sid = lax.axis_index("s")
    wid = sid * NC + cid

    pltpu.sync_copy(ones_s_hbm, acc.at[pl.ds(sid * RPT, RPT)])
    pltpu.sync_copy(ones_c_hbm, rows_v)
    plsc.subcore_barrier()

    def group(g, _):
        g0 = pl.multiple_of(g * GRP, GRP)
        pltpu.sync_copy(dst_hbm.at[wid, pl.ds(g0, GRP)], dst_v)

        def fire(j, _):
            pltpu.async_copy(rows_v, acc.at[dst_v.at[j]], sem, add=True)
            return _
        lax.fori_loop(0, GRP, fire, None, unroll=False)

        def drain(j, _):
            pltpu.make_async_copy(rows_v, acc.at[dst_v.at[j]], sem).wait()
            return _
        lax.fori_loop(0, GRP, drain, None, unroll=False)
        return _
    lax.fori_loop(0, NCHUNK // GRP, group, None, unroll=False)

    plsc.subcore_barrier()
    pltpu.sync_copy(acc.at[pl.ds(sid * RPT, RPT)],
                    out_hbm.at[cid, pl.ds(sid * RPT, RPT)])


DW = 32

_deg_kernel = pl.kernel(
    _deg_body,
    out_type=jax.ShapeDtypeStruct((NC, NPAD, DW), jnp.float32),
    mesh=_sc_mesh(),
    scratch_types=[
        pltpu.VMEM((GRP, CHUNK), jnp.int32),
        pltpu.VMEM((CHUNK, DW), jnp.float32),
        pltpu.VMEM_SHARED((NPAD, DW), jnp.float32),
        pltpu.SemaphoreType.DMA,
    ],
)


def _agg_body(table_hbm, src_hbm, dst_hbm, out_hbm, src_v, dst_v,
              rows0, rows1, acc, sem0, sem1):
    cid = lax.axis_index("c")
    sid = lax.axis_index("s")
    wid = sid * NC + cid

    pltpu.sync_copy(src_hbm.at[wid, pl.ds(0, GRP)], src_v)
    pltpu.sync_copy(dst_hbm.at[wid, pl.ds(0, GRP)], dst_v)
    pltpu.async_copy(table_hbm.at[src_v.at[0]], rows0, sem0)
    pltpu.sync_copy(table_hbm.at[pl.ds(sid * RPT, RPT)],
                    acc.at[pl.ds(sid * RPT, RPT)])
    plsc.subcore_barrier()

    def group(g, _):
        @pl.when(g > 0)
        def _():
            g0 = pl.multiple_of(g * GRP, GRP)
            pltpu.sync_copy(src_hbm.at[wid, pl.ds(g0, GRP)], src_v)
            pltpu.sync_copy(dst_hbm.at[wid, pl.ds(g0, GRP)], dst_v)
            pltpu.async_copy(table_hbm.at[src_v.at[0]], rows0, sem0)

        def pair(k, _):
            j0 = 2 * k
            pltpu.async_copy(table_hbm.at[src_v.at[j0 + 1]], rows1, sem1)
            pltpu.make_async_copy(table_hbm.at[src_v.at[j0]], rows0,
                                  sem0).wait()
            pltpu.sync_copy(rows0, acc.at[dst_v.at[j0]], add=True)

            @pl.when(k < GRP // 2 - 1)
            def _():
                pltpu.async_copy(table_hbm.at[src_v.at[j0 + 2]], rows0, sem0)

            pltpu.make_async_copy(table_hbm.at[src_v.at[j0 + 1]], rows1,
                                  sem1).wait()
            pltpu.sync_copy(rows1, acc.at[dst_v.at[j0 + 1]], add=True)
            return _
        lax.fori_loop(0, GRP // 2, pair, None, unroll=False)
        return _
    lax.fori_loop(0, NCHUNK // GRP, group, None, unroll=False)

    plsc.subcore_barrier()
    pltpu.sync_copy(acc.at[pl.ds(sid * RPT, RPT)],
                    out_hbm.at[cid, pl.ds(sid * RPT, RPT)])


_agg_kernel = pl.kernel(
    _agg_body,
    out_type=jax.ShapeDtypeStruct((NC, NPAD, D), jnp.float32),
    mesh=_sc_mesh(),
    scratch_types=[
        pltpu.VMEM((GRP, CHUNK), jnp.int32),
        pltpu.VMEM((GRP, CHUNK), jnp.int32),
        pltpu.VMEM((CHUNK, D), jnp.float32),
        pltpu.VMEM((CHUNK, D), jnp.float32),
        pltpu.VMEM_SHARED((NPAD, D), jnp.float32),
        pltpu.SemaphoreType.DMA,
        pltpu.SemaphoreType.DMA,
    ],
)


BN = 640


def _mm1_body(x_ref, w_ref, b_ref, d0_ref, d1_ref, s_ref, dinv_ref):
    i = pl.program_id(0)
    col = lax.broadcasted_iota(jnp.int32, (BN, D), 1)
    t = jnp.where(col == 0, 0.0, x_ref[...])
    h = jnp.dot(t, w_ref[...], preferred_element_type=jnp.float32) + b_ref[...]
    row = i * BN + lax.broadcasted_iota(jnp.int32, (BN, D), 0)
    h = jnp.where((col == 0) | (row >= N), 0.0, h)
    deg = d0_ref[:, 0:1] + d1_ref[:, 0:1] - 1.0
    dinv = lax.rsqrt(deg)
    s_ref[...] = h * dinv
    dinv_ref[...] = dinv


def _tc_mm1(xp, W1, b1, deg0, deg1):
    return pl.pallas_call(
        _mm1_body,
        grid=(NPAD // BN,),
        in_specs=[
            pl.BlockSpec((BN, D), lambda i: (i, 0)),
            pl.BlockSpec((D, D), lambda i: (0, 0)),
            pl.BlockSpec((1, D), lambda i: (0, 0)),
            pl.BlockSpec((BN, DW), lambda i: (i, 0)),
            pl.BlockSpec((BN, DW), lambda i: (i, 0)),
        ],
        out_specs=[
            pl.BlockSpec((BN, D), lambda i: (i, 0)),
            pl.BlockSpec((BN, 1), lambda i: (i, 0)),
        ],
        out_shape=[
            jax.ShapeDtypeStruct((NPAD, D), jnp.float32),
            jax.ShapeDtypeStruct((NPAD, 1), jnp.float32),
        ],
    )(xp, W1, b1, deg0, deg1)


def _mm2_body(p0_ref, p1_ref, s_ref, dinv_ref, w_ref, b_ref, out_ref):
    dinv = dinv_ref[...]
    f32 = lambda r: r[...].astype(jnp.float32)
    a = (f32(p0_ref) + f32(p1_ref) - f32(s_ref)) * dinv
    r = jnp.maximum(a, 0.0)
    h = jnp.dot(r, w_ref[...], preferred_element_type=jnp.float32) + b_ref[...]
    col = lax.broadcasted_iota(jnp.int32, (BN, D), 1)
    h = jnp.where(col == 0, 0.0, h)
    out_ref[...] = h * dinv


def _tc_mm2(p0, p1, s1, dinv, W2, b2):
    return pl.pallas_call(
        _mm2_body,
        grid=(NPAD // BN,),
        in_specs=[
            pl.BlockSpec((BN, D), lambda i: (i, 0)),
            pl.BlockSpec((BN, D), lambda i: (i, 0)),
            pl.BlockSpec((BN, D), lambda i: (i, 0)),
            pl.BlockSpec((BN, 1), lambda i: (i, 0)),
            pl.BlockSpec((D, D), lambda i: (0, 0)),
            pl.BlockSpec((1, D), lambda i: (0, 0)),
        ],
        out_specs=pl.BlockSpec((BN, D), lambda i: (i, 0)),
        out_shape=jax.ShapeDtypeStruct((NPAD, D), jnp.float32),
    )(p0, p1, s1, dinv, W2, b2)


def _pool_body(p0_ref, p1_ref, s_ref, dinv_ref, batch_ref, w_ref, b_ref,
               out_ref, pooled_acc, cnt_acc):
    i = pl.program_id(0)
    f32 = lambda r: r[...].astype(jnp.float32)
    a = (f32(p0_ref) + f32(p1_ref) - f32(s_ref)) * dinv_ref[...]
    f = jnp.maximum(a, 0.0)
    onehot = (batch_ref[0] == lax.broadcasted_iota(jnp.int32, (NG, BN), 0)
              ).astype(jnp.float32)
    part = jnp.dot(onehot, f, preferred_element_type=jnp.float32)
    cnt = jnp.sum(onehot, axis=1, keepdims=True)

    @pl.when(i == 0)
    def _():
        pooled_acc[...] = jnp.zeros_like(pooled_acc)
        cnt_acc[...] = jnp.zeros_like(cnt_acc)

    pooled_acc[...] += part
    cnt_acc[...] += cnt

    @pl.when(i == (NPAD // BN) - 1)
    def _():
        pooled = pooled_acc[...] / jnp.maximum(cnt_acc[...], 1.0)
        logits = jnp.dot(pooled, w_ref[...],
                         preferred_element_type=jnp.float32) + b_ref[...]
        m = jnp.max(logits, axis=-1, keepdims=True)
        lse = jnp.log(jnp.sum(jnp.exp(logits - m), axis=-1, keepdims=True)) + m
        out_ref[...] = logits - lse


def _tc_pool(p0, p1, s2, dinv, batch3, W4, b4):
    return pl.pallas_call(
        _pool_body,
        grid=(NPAD // BN,),
        in_specs=[
            pl.BlockSpec((BN, D), lambda i: (i, 0)),
            pl.BlockSpec((BN, D), lambda i: (i, 0)),
            pl.BlockSpec((BN, D), lambda i: (i, 0)),
            pl.BlockSpec((BN, 1), lambda i: (i, 0)),
            pl.BlockSpec((1, 1, BN), lambda i: (i, 0, 0)),
            pl.BlockSpec((D, NCLS), lambda i: (0, 0)),
            pl.BlockSpec((1, NCLS), lambda i: (0, 0)),
        ],
        out_specs=pl.BlockSpec((NG, NCLS), lambda i: (0, 0)),
        out_shape=jax.ShapeDtypeStruct((NG, NCLS), jnp.float32),
        scratch_shapes=[
            pltpu.VMEM((NG, D), jnp.float32),
            pltpu.VMEM((NG, 1), jnp.float32),
        ],
    )(p0, p1, s2, dinv, batch3, W4, b4)


def kernel(x, edge_index, batch, W1, b1, W2, b2, W4, b4):
    e3 = edge_index.reshape(2, NW, NCHUNK, CHUNK)
    src3, dst3 = e3[0], e3[1]
    batch3 = jnp.pad(batch, (0, NPAD - N), constant_values=NG
                     ).reshape(NPAD // BN, 1, BN)
    ones_s = jnp.ones((RPT, DW), jnp.float32)
    ones_c = jnp.ones((CHUNK, DW), jnp.float32)

    degp = _deg_kernel(ones_s, ones_c, dst3)
    s1, dinv = _tc_mm1(x, W1, b1.reshape(1, D), degp[0], degp[1])
    p = _agg_kernel(s1, src3, dst3)
    s2 = _tc_mm2(p[0], p[1], s1, dinv, W2, b2.reshape(1, D))
    p2 = _agg_kernel(s2, src3, dst3)
    return _tc_pool(p2[0], p2[1], s2, dinv, batch3, W4,
                    b4.reshape(1, NCLS))


# speedup vs baseline: 1.1946x; 1.0002x over previous
import functools

import jax
import jax.numpy as jnp
from jax import lax
from jax.experimental import pallas as pl
from jax.experimental.pallas import tpu as pltpu
from jax.experimental.pallas import tpu_sc as plsc

N = 10000
E = 320000
NG = 64
D = 128
NCLS = 10

NC = 2
NS = 16
NW = NC * NS
EPW = E // NW
CHUNK = 125
NCHUNK = EPW // CHUNK
GRP = 40
NPAD = 10240
RPT = NPAD // NS

_sc_mesh = functools.partial(
    plsc.VectorSubcoreMesh, core_axis_name="c", subcore_axis_name="s")


def _deg_body(ones_s_hbm, ones_c_hbm, dst_hbm, out_hbm, dst_v, rows_v, acc,
              sem):
    cid = lax.axis_index("c")
    sid = lax.axis_index("s")
    wid = sid * NC + cid

    pltpu.sync_copy(ones_s_hbm, acc.at[pl.ds(sid * RPT, RPT)])
    pltpu.sync_copy(ones_c_hbm, rows_v)
    plsc.subcore_barrier()

    def group(g, _):
        g0 = pl.multiple_of(g * GRP, GRP)
        pltpu.sync_copy(dst_hbm.at[wid, pl.ds(g0, GRP)], dst_v)

        def fire(j, _):
            pltpu.async_copy(rows_v, acc.at[dst_v.at[j]], sem, add=True)
            return _
        lax.fori_loop(0, GRP, fire, None, unroll=False)

        def drain(j, _):
            pltpu.make_async_copy(rows_v, acc.at[dst_v.at[j]], sem).wait()
            return _
        lax.fori_loop(0, GRP, drain, None, unroll=False)
        return _
    lax.fori_loop(0, NCHUNK // GRP, group, None, unroll=False)

    plsc.subcore_barrier()
    pltpu.sync_copy(acc.at[pl.ds(sid * RPT, RPT)],
                    out_hbm.at[cid, pl.ds(sid * RPT, RPT)])


DW = 32

_deg_kernel = pl.kernel(
    _deg_body,
    out_type=jax.ShapeDtypeStruct((NC, NPAD, DW), jnp.float32),
    mesh=_sc_mesh(),
    scratch_types=[
        pltpu.VMEM((GRP, CHUNK), jnp.int32),
        pltpu.VMEM((CHUNK, DW), jnp.float32),
        pltpu.VMEM_SHARED((NPAD, DW), jnp.float32),
        pltpu.SemaphoreType.DMA,
    ],
)


def _agg_body(table_hbm, src_hbm, dst_hbm, out_hbm, src_v, dst_v,
              rows0, rows1, acc, sem0, sem1):
    cid = lax.axis_index("c")
    sid = lax.axis_index("s")
    wid = sid * NC + cid

    pltpu.sync_copy(src_hbm.at[wid, pl.ds(0, GRP)], src_v)
    pltpu.sync_copy(dst_hbm.at[wid, pl.ds(0, GRP)], dst_v)
    pltpu.async_copy(table_hbm.at[src_v.at[0]], rows0, sem0)
    pltpu.sync_copy(table_hbm.at[pl.ds(sid * RPT, RPT)],
                    acc.at[pl.ds(sid * RPT, RPT)])
    plsc.subcore_barrier()

    def group(g, _):
        @pl.when(g > 0)
        def _():
            g0 = pl.multiple_of(g * GRP, GRP)
            pltpu.sync_copy(src_hbm.at[wid, pl.ds(g0, GRP)], src_v)
            pltpu.sync_copy(dst_hbm.at[wid, pl.ds(g0, GRP)], dst_v)
            pltpu.async_copy(table_hbm.at[src_v.at[0]], rows0, sem0)

        def pair(k, _):
            j0 = 2 * k
            pltpu.async_copy(table_hbm.at[src_v.at[j0 + 1]], rows1, sem1)
            pltpu.make_async_copy(table_hbm.at[src_v.at[j0]], rows0,
                                  sem0).wait()
            pltpu.sync_copy(rows0, acc.at[dst_v.at[j0]], add=True)

            @pl.when(k < GRP // 2 - 1)
            def _():
                pltpu.async_copy(table_hbm.at[src_v.at[j0 + 2]], rows0, sem0)

            pltpu.make_async_copy(table_hbm.at[src_v.at[j0 + 1]], rows1,
                                  sem1).wait()
            pltpu.sync_copy(rows1, acc.at[dst_v.at[j0 + 1]], add=True)
            return _
        lax.fori_loop(0, GRP // 2, pair, None, unroll=False)
        return _
    lax.fori_loop(0, NCHUNK // GRP, group, None, unroll=False)

    plsc.subcore_barrier()
    pltpu.sync_copy(acc.at[pl.ds(sid * RPT, RPT)],
                    out_hbm.at[cid, pl.ds(sid * RPT, RPT)])


_agg_kernel = pl.kernel(
    _agg_body,
    out_type=jax.ShapeDtypeStruct((NC, NPAD, D), jnp.float32),
    mesh=_sc_mesh(),
    scratch_types=[
        pltpu.VMEM((GRP, CHUNK), jnp.int32),
        pltpu.VMEM((GRP, CHUNK), jnp.int32),
        pltpu.VMEM((CHUNK, D), jnp.float32),
        pltpu.VMEM((CHUNK, D), jnp.float32),
        pltpu.VMEM_SHARED((NPAD, D), jnp.float32),
        pltpu.SemaphoreType.DMA,
        pltpu.SemaphoreType.DMA,
    ],
)


BN = 640


def _mm1h_body(x_ref, w_ref, b_ref, h_ref):
    i = pl.program_id(0)
    col = lax.broadcasted_iota(jnp.int32, (BN, D), 1)
    t = jnp.where(col == 0, 0.0, x_ref[...])
    h = jnp.dot(t, w_ref[...], preferred_element_type=jnp.float32) + b_ref[...]
    row = i * BN + lax.broadcasted_iota(jnp.int32, (BN, D), 0)
    h_ref[...] = jnp.where((col == 0) | (row >= N), 0.0, h)


def _tc_mm1h(x, W1, b1):
    return pl.pallas_call(
        _mm1h_body,
        grid=(NPAD // BN,),
        in_specs=[
            pl.BlockSpec((BN, D), lambda i: (i, 0)),
            pl.BlockSpec((D, D), lambda i: (0, 0)),
            pl.BlockSpec((1, D), lambda i: (0, 0)),
        ],
        out_specs=pl.BlockSpec((BN, D), lambda i: (i, 0)),
        out_shape=jax.ShapeDtypeStruct((NPAD, D), jnp.float32),
    )(x, W1, b1)


def _mm1s_body(h_ref, d0_ref, d1_ref, s_ref, dinv_ref):
    deg = d0_ref[:, 0:1] + d1_ref[:, 0:1] - 1.0
    dinv = lax.rsqrt(deg)
    s_ref[...] = h_ref[...] * dinv
    dinv_ref[...] = dinv


def _tc_mm1s(h, deg0, deg1):
    return pl.pallas_call(
        _mm1s_body,
        grid=(NPAD // BN,),
        in_specs=[
            pl.BlockSpec((BN, D), lambda i: (i, 0)),
            pl.BlockSpec((BN, DW), lambda i: (i, 0)),
            pl.BlockSpec((BN, DW), lambda i: (i, 0)),
        ],
        out_specs=[
            pl.BlockSpec((BN, D), lambda i: (i, 0)),
            pl.BlockSpec((BN, 1), lambda i: (i, 0)),
        ],
        out_shape=[
            jax.ShapeDtypeStruct((NPAD, D), jnp.float32),
            jax.ShapeDtypeStruct((NPAD, 1), jnp.float32),
        ],
    )(h, deg0, deg1)


def _mm2_body(p0_ref, p1_ref, s_ref, dinv_ref, w_ref, b_ref, out_ref):
    dinv = dinv_ref[...]
    f32 = lambda r: r[...].astype(jnp.float32)
    a = (f32(p0_ref) + f32(p1_ref) - f32(s_ref)) * dinv
    r = jnp.maximum(a, 0.0)
    h = jnp.dot(r, w_ref[...], preferred_element_type=jnp.float32) + b_ref[...]
    col = lax.broadcasted_iota(jnp.int32, (BN, D), 1)
    h = jnp.where(col == 0, 0.0, h)
    out_ref[...] = h * dinv


def _tc_mm2(p0, p1, s1, dinv, W2, b2):
    return pl.pallas_call(
        _mm2_body,
        grid=(NPAD // BN,),
        in_specs=[
            pl.BlockSpec((BN, D), lambda i: (i, 0)),
            pl.BlockSpec((BN, D), lambda i: (i, 0)),
            pl.BlockSpec((BN, D), lambda i: (i, 0)),
            pl.BlockSpec((BN, 1), lambda i: (i, 0)),
            pl.BlockSpec((D, D), lambda i: (0, 0)),
            pl.BlockSpec((1, D), lambda i: (0, 0)),
        ],
        out_specs=pl.BlockSpec((BN, D), lambda i: (i, 0)),
        out_shape=jax.ShapeDtypeStruct((NPAD, D), jnp.float32),
    )(p0, p1, s1, dinv, W2, b2)


def _pool_body(p0_ref, p1_ref, s_ref, dinv_ref, batch_ref, w_ref, b_ref,
               out_ref, pooled_acc, cnt_acc):
    i = pl.program_id(0)
    f32 = lambda r: r[...].astype(jnp.float32)
    a = (f32(p0_ref) + f32(p1_ref) - f32(s_ref)) * dinv_ref[...]
    f = jnp.maximum(a, 0.0)
    onehot = (batch_ref[0] == lax.broadcasted_iota(jnp.int32, (NG, BN), 0)
              ).astype(jnp.float32)
    part = jnp.dot(onehot, f, preferred_element_type=jnp.float32)
    cnt = jnp.sum(onehot, axis=1, keepdims=True)

    @pl.when(i == 0)
    def _():
        pooled_acc[...] = jnp.zeros_like(pooled_acc)
        cnt_acc[...] = jnp.zeros_like(cnt_acc)

    pooled_acc[...] += part
    cnt_acc[...] += cnt

    @pl.when(i == (NPAD // BN) - 1)
    def _():
        pooled = pooled_acc[...] / jnp.maximum(cnt_acc[...], 1.0)
        logits = jnp.dot(pooled, w_ref[...],
                         preferred_element_type=jnp.float32) + b_ref[...]
        m = jnp.max(logits, axis=-1, keepdims=True)
        lse = jnp.log(jnp.sum(jnp.exp(logits - m), axis=-1, keepdims=True)) + m
        out_ref[...] = logits - lse


def _tc_pool(p0, p1, s2, dinv, batch3, W4, b4):
    return pl.pallas_call(
        _pool_body,
        grid=(NPAD // BN,),
        in_specs=[
            pl.BlockSpec((BN, D), lambda i: (i, 0)),
            pl.BlockSpec((BN, D), lambda i: (i, 0)),
            pl.BlockSpec((BN, D), lambda i: (i, 0)),
            pl.BlockSpec((BN, 1), lambda i: (i, 0)),
            pl.BlockSpec((1, 1, BN), lambda i: (i, 0, 0)),
            pl.BlockSpec((D, NCLS), lambda i: (0, 0)),
            pl.BlockSpec((1, NCLS), lambda i: (0, 0)),
        ],
        out_specs=pl.BlockSpec((NG, NCLS), lambda i: (0, 0)),
        out_shape=jax.ShapeDtypeStruct((NG, NCLS), jnp.float32),
        scratch_shapes=[
            pltpu.VMEM((NG, D), jnp.float32),
            pltpu.VMEM((NG, 1), jnp.float32),
        ],
    )(p0, p1, s2, dinv, batch3, W4, b4)


def kernel(x, edge_index, batch, W1, b1, W2, b2, W4, b4):
    e3 = edge_index.reshape(2, NW, NCHUNK, CHUNK)
    src3, dst3 = e3[0], e3[1]
    batch3 = jnp.pad(batch, (0, NPAD - N), constant_values=NG
                     ).reshape(NPAD // BN, 1, BN)
    ones_s = jnp.ones((RPT, DW), jnp.float32)
    ones_c = jnp.ones((CHUNK, DW), jnp.float32)

    degp = _deg_kernel(ones_s, ones_c, dst3)
    h1 = _tc_mm1h(x, W1, b1.reshape(1, D))
    s1, dinv = _tc_mm1s(h1, degp[0], degp[1])
    p = _agg_kernel(s1, src3, dst3)
    s2 = _tc_mm2(p[0], p[1], s1, dinv, W2, b2.reshape(1, D))
    p2 = _agg_kernel(s2, src3, dst3)
    return _tc_pool(p2[0], p2[1], s2, dinv, batch3, W4,
                    b4.reshape(1, NCLS))
